# baseline (speedup 1.0000x reference)
import functools

import jax
import jax.numpy as jnp
from jax.experimental import pallas as pl
from jax.experimental.pallas import tpu as pltpu

BM = 256
BK = 2048


def _mm_kernel(t_ref, x_ref, o_ref):
    k = pl.program_id(1)

    @pl.when(k == 0)
    def _():
        o_ref[...] = jnp.zeros_like(o_ref)

    t = t_ref[...].astype(jnp.bfloat16)
    xk = x_ref[pl.ds(k * BK, BK), :]
    o_ref[...] += jnp.dot(t, xk, preferred_element_type=jnp.float32)


@jax.jit
def kernel(x, transform):
    B, V_in, C = x.shape
    V_out = transform.shape[0]
    N = B * C
    xt = x.astype(jnp.bfloat16).transpose(1, 0, 2).reshape(V_in, N)

    out2d = pl.pallas_call(
        _mm_kernel,
        grid=(V_out // BM, V_in // BK),
        in_specs=[
            pl.BlockSpec((BM, BK), lambda m, k: (m, k)),
            pl.BlockSpec((V_in, N), lambda m, k: (0, 0)),
        ],
        out_specs=pl.BlockSpec((BM, N), lambda m, k: (m, 0)),
        out_shape=jax.ShapeDtypeStruct((V_out, N), jnp.float32),
        compiler_params=pltpu.CompilerParams(
            dimension_semantics=("parallel", "arbitrary"),
        ),
    )(transform, xt)

    return out2d.reshape(V_out, B, C).transpose(1, 0, 2)
